# Initial kernel scaffold; baseline (speedup 1.0000x reference)
import jax
import jax.numpy as jnp
from jax.experimental import pallas as pl


def kernel(node_feats, edge_index, edge_attr, W, att, bias, bn_node_g, bn_node_b, Wf, bf, Ws, bs, bn_edge_g, bn_edge_b):
    raise NotImplementedError("write your pallas kernel here")


# speedup vs baseline: 13.0653x; 13.0653x over previous
import functools

import jax
import jax.numpy as jnp
from jax import lax
from jax.experimental import pallas as pl
from jax.experimental.pallas import tpu as pltpu
from jax.experimental.pallas import tpu_sc as plsc

NN = 10000
EE = 320000
DIM = 128
HEADS = 4

NC, NS = 2, 16
NW = NC * NS
EW = EE // NW
CH = 80
ITERS = EW // CH

EB = 2560
GE = EE // EB
NB = 2000
GN = NN // NB


def _silu(x):
    return x * jax.nn.sigmoid(x)


def _sc_mesh():
    return plsc.VectorSubcoreMesh(core_axis_name="c", subcore_axis_name="s",
                                  num_cores=NC, num_subcores=NS)


def _gather_pair(table, iarr, jarr, d):
    out = jax.ShapeDtypeStruct((EE, d), jnp.float32)

    @functools.partial(
        pl.kernel,
        out_type=(out, out),
        mesh=_sc_mesh(),
        scratch_types=[
            pltpu.VMEM((CH,), jnp.int32),
            pltpu.VMEM((CH,), jnp.int32),
            pltpu.VMEM((CH, d), jnp.float32),
            pltpu.VMEM((CH, d), jnp.float32),
            pltpu.SemaphoreType.DMA,
            pltpu.SemaphoreType.DMA,
        ],
    )
    def k(table_hbm, i_hbm, j_hbm, oi_hbm, oj_hbm, iv, jv, ri, rj, s1, s2):
        wid = lax.axis_index("s") * NC + lax.axis_index("c")

        def body(t, carry):
            base = wid * EW + t * CH
            pltpu.sync_copy(i_hbm.at[pl.ds(base, CH)], iv)
            pltpu.sync_copy(j_hbm.at[pl.ds(base, CH)], jv)
            c1 = pltpu.async_copy(table_hbm.at[iv], ri, s1)
            c2 = pltpu.async_copy(table_hbm.at[jv], rj, s2)
            c1.wait()
            c2.wait()
            pltpu.sync_copy(ri, oi_hbm.at[pl.ds(base, CH)])
            pltpu.sync_copy(rj, oj_hbm.at[pl.ds(base, CH)])
            return carry

        lax.fori_loop(0, ITERS, body, 0, unroll=False)

    return k(table, iarr, jarr)


def _gather_one(table, iarr, d):
    @functools.partial(
        pl.kernel,
        out_type=jax.ShapeDtypeStruct((EE, d), jnp.float32),
        mesh=_sc_mesh(),
        compiler_params=pltpu.CompilerParams(use_tc_tiling_on_sc=False),
        scratch_types=[
            pltpu.VMEM((CH,), jnp.int32),
            pltpu.VMEM((CH, d), jnp.float32),
            pltpu.SemaphoreType.DMA,
        ],
    )
    def k(table_hbm, i_hbm, oi_hbm, iv, ri, s1):
        wid = lax.axis_index("s") * NC + lax.axis_index("c")

        def body(t, carry):
            base = wid * EW + t * CH
            pltpu.sync_copy(i_hbm.at[pl.ds(base, CH)], iv)
            pltpu.async_copy(table_hbm.at[iv], ri, s1).wait()
            pltpu.sync_copy(ri, oi_hbm.at[pl.ds(base, CH)])
            return carry

        lax.fori_loop(0, ITERS, body, 0, unroll=False)

    return k(table, iarr)


def _scatter_add(vals, iarr, d):
    zeros = jnp.zeros((NN, d), jnp.float32)

    @functools.partial(
        pl.kernel,
        out_type=jax.ShapeDtypeStruct((NC, NN, d), jnp.float32),
        mesh=_sc_mesh(),
        compiler_params=pltpu.CompilerParams(use_tc_tiling_on_sc=False),
        scratch_types=[
            pltpu.VMEM((1, CH), jnp.int32),
            pltpu.VMEM((CH, d), jnp.float32),
            pltpu.VMEM_SHARED((NN, d), jnp.float32),
        ],
    )
    def k(vals_hbm, i_hbm, z_hbm, out_hbm, iv, vv, acc):
        cid = lax.axis_index("c")
        sid = lax.axis_index("s")
        wid = sid * NC + cid

        @pl.when(sid == 0)
        def _():
            pltpu.sync_copy(z_hbm, acc)

        plsc.subcore_barrier()

        def body(t, carry):
            base = wid * EW + t * CH
            pltpu.sync_copy(i_hbm.at[pl.ds(base, CH)], iv.at[0])
            pltpu.sync_copy(vals_hbm.at[pl.ds(base, CH)], vv)
            pltpu.sync_copy(vv, acc.at[iv.at[0]], add=True)
            return carry

        lax.fori_loop(0, ITERS, body, 0, unroll=False)
        plsc.subcore_barrier()

        @pl.when(sid == 0)
        def _():
            pltpu.sync_copy(acc, out_hbm.at[cid])

    return k(vals, iarr, zeros)


def _pass1_body(xi, xj, ea, w, a1, a2, alpha_ref, njb_ref, ssum_ref, smax_ref):
    e = pl.program_id(0)
    wx = w[0:DIM, :]
    we = w[DIM:2 * DIM, :]
    eaw = jnp.dot(ea[...], we, preferred_element_type=jnp.float32)
    ni = _silu(jnp.dot(xi[...], wx, preferred_element_type=jnp.float32) + eaw)
    nj = _silu(jnp.dot(xj[...], wx, preferred_element_type=jnp.float32) + eaw)
    njb_ref[...] = nj.astype(jnp.bfloat16)
    cols = []
    for h in range(HEADS):
        sl = slice(h * DIM, (h + 1) * DIM)
        a_h = jnp.sum(ni[:, sl] * a1[h, :][None, :]
                      + nj[:, sl] * a2[h, :][None, :], axis=1)
        cols.append(a_h)
    a = _silu(jnp.stack(cols, axis=1))
    alpha_ref[...] = a
    s1 = jnp.sum(a, axis=0)
    s2 = jnp.sum(a * a, axis=0)
    mx = jnp.max(a, axis=0)
    mn = jnp.min(a, axis=0)
    padz = jnp.zeros((128 - HEADS,), jnp.float32)
    row = lambda v: jnp.concatenate([v, padz])[None, :]
    blk_sum = jnp.concatenate(
        [row(s1), row(s2), jnp.zeros((6, 128), jnp.float32)], axis=0)
    padm = jnp.full((128 - HEADS,), -1e30, jnp.float32)
    rowm = lambda v: jnp.concatenate([v, padm])[None, :]
    blk_max = jnp.concatenate(
        [rowm(mx), rowm(-mn), jnp.full((6, 128), -1e30, jnp.float32)], axis=0)

    @pl.when(e == 0)
    def _():
        ssum_ref[...] = jnp.zeros_like(ssum_ref)
        smax_ref[...] = jnp.full_like(smax_ref, -1e30)

    ssum_ref[...] = ssum_ref[...] + blk_sum
    smax_ref[...] = jnp.maximum(smax_ref[...], blk_max)


def _ex_body(alpha, prm, ex_ref):
    a = alpha[...]
    scale = prm[0:1, :HEADS]
    shift = prm[1:2, :HEADS]
    u = prm[2:3, :HEADS]
    ab = _silu(a * scale + shift)
    ex_ref[...] = jnp.concatenate(
        [jnp.exp(ab - u), jnp.zeros((a.shape[0], 16 - HEADS), jnp.float32)],
        axis=1)


def _den_body(denp, inv_ref):
    d = denp[0] + denp[1]
    inv_ref[...] = 1.0 / (d + 1e-16)


def _msg_body(njb, ex, deni, msg_ref):
    exv = ex[...]
    dv = deni[...]
    acc = jnp.zeros((EB, DIM), jnp.float32)
    for h in range(HEADS):
        w_h = exv[:, h] * dv[:, h]
        nj_h = njb[:, h * DIM:(h + 1) * DIM].astype(jnp.float32)
        acc = acc + nj_h * w_h[:, None]
    msg_ref[...] = acc * (1.0 / HEADS)


def _node_body(nf, p0, p1, bias, out_ref):
    out_ref[...] = nf[...] + p0[...] + p1[...] + bias[0:1, :]


def _edge_body(ni, nj, ea, wf, ws, bfb, bsb, m_ref, stat_ref):
    e = pl.program_id(0)
    niv = ni[...]
    njv = nj[...]
    eav = ea[...]
    zf = (jnp.dot(niv, wf[0:DIM, :], preferred_element_type=jnp.float32)
          + jnp.dot(njv, wf[DIM:2 * DIM, :], preferred_element_type=jnp.float32)
          + jnp.dot(eav, wf[2 * DIM:3 * DIM, :], preferred_element_type=jnp.float32)
          + bfb[0:1, :])
    zs = (jnp.dot(niv, ws[0:DIM, :], preferred_element_type=jnp.float32)
          + jnp.dot(njv, ws[DIM:2 * DIM, :], preferred_element_type=jnp.float32)
          + jnp.dot(eav, ws[2 * DIM:3 * DIM, :], preferred_element_type=jnp.float32)
          + bsb[0:1, :])
    m = jax.nn.sigmoid(zf) * jax.nn.softplus(zs)
    m_ref[...] = m.astype(jnp.bfloat16)
    s1 = jnp.sum(m, axis=0)
    s2 = jnp.sum(m * m, axis=0)
    blk = jnp.concatenate([s1[None, :], s2[None, :],
                           jnp.zeros((6, DIM), jnp.float32)], axis=0)

    @pl.when(e == 0)
    def _():
        stat_ref[...] = jnp.zeros_like(stat_ref)

    stat_ref[...] = stat_ref[...] + blk


def _edge_out_body(ea, m, sc, sh, out_ref):
    out_ref[...] = ea[...] + m[...].astype(jnp.float32) * sc[0:1, :] + sh[0:1, :]


def _pass1(xi, xj, ea, w, a1, a2):
    return pl.pallas_call(
        _pass1_body,
        grid=(GE,),
        in_specs=[
            pl.BlockSpec((EB, DIM), lambda e: (e, 0)),
            pl.BlockSpec((EB, DIM), lambda e: (e, 0)),
            pl.BlockSpec((EB, DIM), lambda e: (e, 0)),
            pl.BlockSpec((2 * DIM, HEADS * DIM), lambda e: (0, 0)),
            pl.BlockSpec((HEADS, DIM), lambda e: (0, 0)),
            pl.BlockSpec((HEADS, DIM), lambda e: (0, 0)),
        ],
        out_specs=[
            pl.BlockSpec((EB, HEADS), lambda e: (e, 0)),
            pl.BlockSpec((EB, HEADS * DIM), lambda e: (e, 0)),
            pl.BlockSpec((8, 128), lambda e: (0, 0)),
            pl.BlockSpec((8, 128), lambda e: (0, 0)),
        ],
        out_shape=[
            jax.ShapeDtypeStruct((EE, HEADS), jnp.float32),
            jax.ShapeDtypeStruct((EE, HEADS * DIM), jnp.bfloat16),
            jax.ShapeDtypeStruct((8, 128), jnp.float32),
            jax.ShapeDtypeStruct((8, 128), jnp.float32),
        ],
    )(xi, xj, ea, w, a1, a2)


def _ex(alpha, prm):
    return pl.pallas_call(
        _ex_body,
        grid=(GE,),
        in_specs=[
            pl.BlockSpec((EB, HEADS), lambda e: (e, 0)),
            pl.BlockSpec((8, 128), lambda e: (0, 0)),
        ],
        out_specs=pl.BlockSpec((EB, 16), lambda e: (e, 0)),
        out_shape=jax.ShapeDtypeStruct((EE, 16), jnp.float32),
    )(alpha, prm)


def _den(denp):
    return pl.pallas_call(
        _den_body,
        out_shape=jax.ShapeDtypeStruct((NN, 16), jnp.float32),
    )(denp)


def _msg(njb, ex, deni):
    return pl.pallas_call(
        _msg_body,
        grid=(GE,),
        in_specs=[
            pl.BlockSpec((EB, HEADS * DIM), lambda e: (e, 0)),
            pl.BlockSpec((EB, 16), lambda e: (e, 0)),
            pl.BlockSpec((EB, 16), lambda e: (e, 0)),
        ],
        out_specs=pl.BlockSpec((EB, DIM), lambda e: (e, 0)),
        out_shape=jax.ShapeDtypeStruct((EE, DIM), jnp.float32),
    )(njb, ex, deni)


def _node(nf, p0, p1, bias):
    return pl.pallas_call(
        _node_body,
        grid=(GN,),
        in_specs=[
            pl.BlockSpec((NB, DIM), lambda n: (n, 0)),
            pl.BlockSpec((NB, DIM), lambda n: (n, 0)),
            pl.BlockSpec((NB, DIM), lambda n: (n, 0)),
            pl.BlockSpec((8, 128), lambda n: (0, 0)),
        ],
        out_specs=pl.BlockSpec((NB, DIM), lambda n: (n, 0)),
        out_shape=jax.ShapeDtypeStruct((NN, DIM), jnp.float32),
    )(nf, p0, p1, bias)


def _edge(ni, nj, ea, wf, ws, bfb, bsb):
    return pl.pallas_call(
        _edge_body,
        grid=(GE,),
        in_specs=[
            pl.BlockSpec((EB, DIM), lambda e: (e, 0)),
            pl.BlockSpec((EB, DIM), lambda e: (e, 0)),
            pl.BlockSpec((EB, DIM), lambda e: (e, 0)),
            pl.BlockSpec((3 * DIM, DIM), lambda e: (0, 0)),
            pl.BlockSpec((3 * DIM, DIM), lambda e: (0, 0)),
            pl.BlockSpec((8, 128), lambda e: (0, 0)),
            pl.BlockSpec((8, 128), lambda e: (0, 0)),
        ],
        out_specs=[
            pl.BlockSpec((EB, DIM), lambda e: (e, 0)),
            pl.BlockSpec((8, 128), lambda e: (0, 0)),
        ],
        out_shape=[
            jax.ShapeDtypeStruct((EE, DIM), jnp.bfloat16),
            jax.ShapeDtypeStruct((8, 128), jnp.float32),
        ],
    )(ni, nj, ea, wf, ws, bfb, bsb)


def _edge_out(ea, m, sc, sh):
    return pl.pallas_call(
        _edge_out_body,
        grid=(GE,),
        in_specs=[
            pl.BlockSpec((EB, DIM), lambda e: (e, 0)),
            pl.BlockSpec((EB, DIM), lambda e: (e, 0)),
            pl.BlockSpec((8, 128), lambda e: (0, 0)),
            pl.BlockSpec((8, 128), lambda e: (0, 0)),
        ],
        out_specs=pl.BlockSpec((EB, DIM), lambda e: (e, 0)),
        out_shape=jax.ShapeDtypeStruct((EE, DIM), jnp.float32),
    )(ea, m, sc, sh)


def _pad8(v):
    return jnp.zeros((8, 128), jnp.float32).at[0, :v.shape[0]].set(v)


def kernel(node_feats, edge_index, edge_attr, W, att, bias, bn_node_g,
           bn_node_b, Wf, bf, Ws, bs, bn_edge_g, bn_edge_b):
    i = edge_index[0]
    j = edge_index[1]
    att2d = att.reshape(HEADS, 2 * DIM)
    a1 = att2d[:, :DIM]
    a2 = att2d[:, DIM:]

    xi, xj = _gather_pair(node_feats, i, j, DIM)

    alpha_raw, njb, ssum, smax = _pass1(xi, xj, edge_attr, W, a1, a2)

    e_f = float(EE)
    s1 = ssum[0, :HEADS]
    s2 = ssum[1, :HEADS]
    mean = s1 / e_f
    var = s2 / e_f - mean * mean
    scale = bn_node_g / jnp.sqrt(var + 1e-5)
    shift = bn_node_b - mean * scale
    zx = smax[0, :HEADS] * scale + shift
    zn = (-smax[1, :HEADS]) * scale + shift
    u = jnp.maximum(_silu(zx), _silu(zn))
    prm = jnp.zeros((8, 128), jnp.float32)
    prm = prm.at[0, :HEADS].set(scale).at[1, :HEADS].set(shift).at[2, :HEADS].set(u)

    ex = _ex(alpha_raw, prm)

    denp = _scatter_add(ex, i, 16)
    invden = _den(denp)
    deni = _gather_one(invden, i, 16)

    msg = _msg(njb, ex, deni)
    nodep = _scatter_add(msg, i, DIM)
    node_feat = _node(node_feats, nodep[0], nodep[1], _pad8(bias))

    ni2, nj2 = _gather_pair(node_feat, i, j, DIM)
    m_bf, stat = _edge(ni2, nj2, edge_attr, Wf, Ws, _pad8(bf), _pad8(bs))
    mean2 = stat[0] / e_f
    var2 = stat[1] / e_f - mean2 * mean2
    scale2 = bn_edge_g / jnp.sqrt(var2 + 1e-5)
    shift2 = bn_edge_b - mean2 * scale2
    edge_feat = _edge_out(edge_attr, m_bf, _pad8(scale2), _pad8(shift2))

    return node_feat, edge_feat
